# Initial kernel scaffold; baseline (speedup 1.0000x reference)
import jax
import jax.numpy as jnp
from jax.experimental import pallas as pl


def kernel(hidden_states, ln_weight, ln_bias, k_weight, q_weight, bias):
    raise NotImplementedError("write your pallas kernel here")


# speedup vs baseline: 2.1108x; 2.1108x over previous
import jax
import jax.numpy as jnp
from jax.experimental import pallas as pl
from jax.experimental.pallas import tpu as pltpu

_HASHCODE_LEN = 9
_LN_EPS = 1e-12

_C0 = 1.0 / jnp.pi
_C1 = (1.0 / 6.0) / jnp.pi
_C2 = (3.0 / 40.0) / jnp.pi
_C3 = (15.0 / 336.0) / jnp.pi
_C4 = (105.0 / 3456.0) / jnp.pi


def _hash_weight(s):
    s2 = s * s
    t = _C0 + s2 * (_C1 + s2 * (_C2 + s2 * (_C3 + s2 * _C4)))
    u = 0.5 + s * t
    u2 = u * u
    u4 = u2 * u2
    u8 = u4 * u4
    return u8 * u


def _yoso_body(x_ref, kwt_ref, qw_ref, lnw_ref, lnb_ref, bias_ref, out_ref,
               ksc_ref):
    i = pl.program_id(0)

    @pl.when(i == 0)
    def _init():
        kw = kwt_ref[...].astype(jnp.float32)
        inv_kn = jax.lax.rsqrt(
            jnp.maximum(jnp.sum(kw * kw, axis=0, keepdims=True), 1e-24))
        ksc_ref[...] = (kw * inv_kn).astype(jnp.bfloat16)

    x = x_ref[...]
    mean = jnp.mean(x, axis=-1, keepdims=True)
    xc = x - mean
    var = jnp.mean(xc * xc, axis=-1, keepdims=True)
    xn = xc * jax.lax.rsqrt(var + _LN_EPS)
    xn = xn * lnw_ref[...] + lnb_ref[...]
    q = xn * jax.lax.rsqrt(jnp.maximum(jnp.sum(xn * xn, axis=-1, keepdims=True), 1e-24))
    qb = q.astype(jnp.bfloat16)
    s = jax.lax.dot_general(qb, ksc_ref[...], (((1,), (0,)), ((), ())),
                            preferred_element_type=jnp.float32)
    w = _hash_weight(s).astype(jnp.bfloat16)
    xo = jax.lax.dot_general(w, qw_ref[...], (((1,), (0,)), ((), ())),
                             preferred_element_type=jnp.float32)
    xo = xo * jax.lax.rsqrt(jnp.maximum(jnp.sum(xo * xo, axis=-1, keepdims=True), 1e-24))
    out_ref[...] = xo + bias_ref[...]


def _yoso(x, kwt, qw, lnw, lnb, bias, block_m):
    n, h = x.shape
    inter = kwt.shape[1]
    grid = (n // block_m,)
    return pl.pallas_call(
        _yoso_body,
        grid=grid,
        in_specs=[
            pl.BlockSpec((block_m, h), lambda i: (i, 0)),
            pl.BlockSpec((h, inter), lambda i: (0, 0)),
            pl.BlockSpec((inter, h), lambda i: (0, 0)),
            pl.BlockSpec((1, h), lambda i: (0, 0)),
            pl.BlockSpec((1, h), lambda i: (0, 0)),
            pl.BlockSpec((1, h), lambda i: (0, 0)),
        ],
        out_specs=pl.BlockSpec((block_m, h), lambda i: (i, 0)),
        out_shape=jax.ShapeDtypeStruct((n, h), jnp.float32),
        scratch_shapes=[pltpu.VMEM((h, inter), jnp.bfloat16)],
        compiler_params=pltpu.CompilerParams(
            dimension_semantics=("arbitrary",),
            vmem_limit_bytes=100 * 1024 * 1024,
        ),
    )(x, kwt, qw, lnw, lnb, bias)


def kernel(hidden_states, ln_weight, ln_bias, k_weight, q_weight, bias):
    shape = hidden_states.shape[:-1]
    h = hidden_states.shape[-1]
    x = hidden_states.reshape(-1, h)
    kwt = k_weight.T.astype(jnp.bfloat16)
    qw = q_weight.astype(jnp.bfloat16)
    out = _yoso(x, kwt, qw,
                ln_weight.reshape(1, h), ln_bias.reshape(1, h),
                bias.reshape(1, h), block_m=256)
    return out.reshape(*shape, h)
